# Initial kernel scaffold; baseline (speedup 1.0000x reference)
import jax
import jax.numpy as jnp
from jax.experimental import pallas as pl


def kernel(data, x_indices, y_indices):
    raise NotImplementedError("write your pallas kernel here")


# speedup vs baseline: 1.6216x; 1.6216x over previous
import functools

import jax
import jax.numpy as jnp
from jax import lax
from jax.experimental import pallas as pl
from jax.experimental.pallas import tpu as pltpu
from jax.experimental.pallas import tpu_sc as plsc

_B, _H, _W = 256, 64, 1024
_HW = _H * _W
_N = 50000
_NW = 32
_BPW = _B // _NW
_CHUNK = 2000
_NCHUNK = _N // _CHUNK
_VPC = _CHUNK // 16


def _make_kernel():
    mesh = plsc.VectorSubcoreMesh(core_axis_name="c", subcore_axis_name="s")

    @functools.partial(
        pl.kernel,
        mesh=mesh,
        out_type=jax.ShapeDtypeStruct((_B * _N,), jnp.float32),
        compiler_params=pltpu.CompilerParams(needs_layout_passes=False),
        scratch_types=[
            pltpu.VMEM((_N,), jnp.int32),
            pltpu.VMEM((_HW,), jnp.float32),
            pltpu.VMEM((_CHUNK,), jnp.float32),
            pltpu.VMEM((_CHUNK,), jnp.int32),
            pltpu.VMEM((_CHUNK,), jnp.int32),
        ],
    )
    def gather_kernel(x_hbm, y_hbm, data_hbm, out_hbm,
                      idx_v, row_v, out_v, xt_v, yt_v):
        wid = lax.axis_index("s") * 2 + lax.axis_index("c")

        def build_chunk(c, carry):
            base = pl.multiple_of(c * _CHUNK, _CHUNK)
            pltpu.sync_copy(x_hbm.at[pl.ds(base, _CHUNK)], xt_v)
            pltpu.sync_copy(y_hbm.at[pl.ds(base, _CHUNK)], yt_v)
            for j in range(_VPC):
                xv = xt_v[pl.ds(j * 16, 16)]
                yv = yt_v[pl.ds(j * 16, 16)]
                idx_v[pl.ds(base + j * 16, 16)] = xv * _W + yv
            return carry

        lax.fori_loop(0, _NCHUNK, build_chunk, None)

        def do_batch(i, carry):
            b = wid * _BPW + i
            row_off = pl.multiple_of(b * _HW, 8)
            out_off = pl.multiple_of(b * _N, 8)
            pltpu.sync_copy(data_hbm.at[pl.ds(row_off, _HW)], row_v)

            def do_chunk(c, carry2):
                base = pl.multiple_of(c * _CHUNK, _CHUNK)
                for j in range(_VPC):
                    iv = idx_v[pl.ds(base + j * 16, 16)]
                    out_v[pl.ds(j * 16, 16)] = plsc.load_gather(row_v, [iv])
                pltpu.sync_copy(out_v, out_hbm.at[pl.ds(out_off + base, _CHUNK)])
                return carry2

            lax.fori_loop(0, _NCHUNK, do_chunk, None)
            return carry

        lax.fori_loop(0, _BPW, do_batch, None)

    return gather_kernel


_gather = _make_kernel()


def kernel(data, x_indices, y_indices):
    B, H, W = data.shape
    x = x_indices.astype(jnp.int32)
    y = y_indices.astype(jnp.int32)
    out = _gather(x, y, data.reshape(B * H * W))
    return out.reshape(B, _N)


# speedup vs baseline: 1.7108x; 1.0550x over previous
import functools

import jax
import jax.numpy as jnp
from jax import lax
from jax.experimental import pallas as pl
from jax.experimental.pallas import tpu as pltpu
from jax.experimental.pallas import tpu_sc as plsc

_B, _H, _W = 256, 64, 1024
_HW = _H * _W
_N = 50000
_NW = 32
_BPW = _B // _NW
_CHUNK = 2000
_NCHUNK = _N // _CHUNK
_NPAIR = (_NCHUNK - 1) // 2
_VPC = _CHUNK // 16


def _make_kernel():
    mesh = plsc.VectorSubcoreMesh(core_axis_name="c", subcore_axis_name="s")

    @functools.partial(
        pl.kernel,
        mesh=mesh,
        out_type=jax.ShapeDtypeStruct((_B * _N,), jnp.float32),
        compiler_params=pltpu.CompilerParams(needs_layout_passes=False),
        scratch_types=[
            pltpu.VMEM((_N,), jnp.int32),
            pltpu.VMEM((_HW,), jnp.float32),
            pltpu.VMEM((_CHUNK,), jnp.float32),
            pltpu.VMEM((_CHUNK,), jnp.float32),
            pltpu.VMEM((_CHUNK,), jnp.int32),
            pltpu.VMEM((_CHUNK,), jnp.int32),
            pltpu.SemaphoreType.DMA,
            pltpu.SemaphoreType.DMA,
        ],
    )
    def gather_kernel(x_hbm, y_hbm, data_hbm, out_hbm,
                      idx_v, row_v, out0_v, out1_v, xt_v, yt_v, sem0, sem1):
        wid = lax.axis_index("s") * 2 + lax.axis_index("c")

        def build_chunk(c, carry):
            base = pl.multiple_of(c * _CHUNK, _CHUNK)
            pltpu.sync_copy(x_hbm.at[pl.ds(base, _CHUNK)], xt_v)
            pltpu.sync_copy(y_hbm.at[pl.ds(base, _CHUNK)], yt_v)
            for j in range(_VPC):
                xv = xt_v[pl.ds(j * 16, 16)]
                yv = yt_v[pl.ds(j * 16, 16)]
                idx_v[pl.ds(base + j * 16, 16)] = xv * _W + yv
            return carry

        lax.fori_loop(0, _NCHUNK, build_chunk, None)

        def gather_chunk(buf, cbase):
            for j in range(_VPC):
                iv = idx_v[pl.ds(cbase + j * 16, 16)]
                buf[pl.ds(j * 16, 16)] = plsc.load_gather(row_v, [iv])

        def do_batch(i, carry):
            b = wid * _BPW + i
            row_off = pl.multiple_of(b * _HW, 8)
            out_off = pl.multiple_of(b * _N, 8)
            pltpu.sync_copy(data_hbm.at[pl.ds(row_off, _HW)], row_v)

            def pair(t, carry2):
                base0 = pl.multiple_of(t * 2 * _CHUNK, _CHUNK)
                base1 = base0 + _CHUNK
                dst0 = out_hbm.at[pl.ds(out_off + base0, _CHUNK)]
                dst1 = out_hbm.at[pl.ds(out_off + base1, _CHUNK)]

                @pl.when(t > 0)
                def _wait0():
                    pltpu.make_async_copy(out0_v, dst0, sem0).wait()

                gather_chunk(out0_v, base0)
                pltpu.async_copy(out0_v, dst0, sem0)

                @pl.when(t > 0)
                def _wait1():
                    pltpu.make_async_copy(out1_v, dst1, sem1).wait()

                gather_chunk(out1_v, base1)
                pltpu.async_copy(out1_v, dst1, sem1)
                return carry2

            lax.fori_loop(0, _NPAIR, pair, None)

            tail = pl.multiple_of(2 * _NPAIR * _CHUNK, _CHUNK)
            dst_t = out_hbm.at[pl.ds(out_off + tail, _CHUNK)]
            pltpu.make_async_copy(out0_v, dst_t, sem0).wait()
            pltpu.make_async_copy(out1_v, dst_t, sem1).wait()
            gather_chunk(out0_v, tail)
            pltpu.sync_copy(out0_v, dst_t)
            return carry

        lax.fori_loop(0, _BPW, do_batch, None)

    return gather_kernel


_gather = _make_kernel()


def kernel(data, x_indices, y_indices):
    B, H, W = data.shape
    x = x_indices.astype(jnp.int32)
    y = y_indices.astype(jnp.int32)
    out = _gather(x, y, data.reshape(B * H * W))
    return out.reshape(B, _N)


# speedup vs baseline: 2.7801x; 1.6250x over previous
import functools

import jax
import jax.numpy as jnp
from jax import lax
from jax.experimental import pallas as pl
from jax.experimental.pallas import tpu as pltpu
from jax.experimental.pallas import tpu_sc as plsc

_B, _H, _W = 256, 64, 1024
_HW = _H * _W
_N = 50000
_NW = 32
_BPW = _B // _NW
_CHUNK = 2000
_NCHUNK = _N // _CHUNK
_NPAIR = (_NCHUNK - 1) // 2
_VPC = _CHUNK // 16


def _make_kernel():
    mesh = plsc.VectorSubcoreMesh(core_axis_name="c", subcore_axis_name="s")

    @functools.partial(
        pl.kernel,
        mesh=mesh,
        out_type=jax.ShapeDtypeStruct((_B * _N,), jnp.float32),
        compiler_params=pltpu.CompilerParams(needs_layout_passes=False),
        scratch_types=[
            pltpu.VMEM((_N,), jnp.int32),
            pltpu.VMEM((_HW,), jnp.float32),
            pltpu.VMEM((_CHUNK,), jnp.float32),
            pltpu.VMEM((_CHUNK,), jnp.float32),
            pltpu.VMEM((_CHUNK,), jnp.int32),
            pltpu.VMEM((_CHUNK,), jnp.int32),
            pltpu.SemaphoreType.DMA,
            pltpu.SemaphoreType.DMA,
        ],
    )
    def gather_kernel(x_hbm, y_hbm, data_hbm, out_hbm,
                      idx_v, row_v, out0_v, out1_v, xt_v, yt_v, sem0, sem1):
        wid = lax.axis_index("s") * 2 + lax.axis_index("c")

        def build_chunk(c, carry):
            base = pl.multiple_of(c * _CHUNK, _CHUNK)
            pltpu.sync_copy(x_hbm.at[pl.ds(base, _CHUNK)], xt_v)
            pltpu.sync_copy(y_hbm.at[pl.ds(base, _CHUNK)], yt_v)

            @plsc.parallel_loop(0, _VPC, unroll=8)
            def _build(j):
                xv = xt_v[pl.ds(j * 16, 16)]
                yv = yt_v[pl.ds(j * 16, 16)]
                idx_v[pl.ds(base + j * 16, 16)] = xv * _W + yv

            return carry

        lax.fori_loop(0, _NCHUNK, build_chunk, None)

        def gather_chunk(buf, cbase):
            @plsc.parallel_loop(0, _VPC, unroll=8)
            def _gather(j):
                iv = idx_v[pl.ds(cbase + j * 16, 16)]
                buf[pl.ds(j * 16, 16)] = plsc.load_gather(row_v, [iv])

        def do_batch(i, carry):
            b = wid * _BPW + i
            row_off = pl.multiple_of(b * _HW, 8)
            out_off = pl.multiple_of(b * _N, 8)
            pltpu.sync_copy(data_hbm.at[pl.ds(row_off, _HW)], row_v)

            def pair(t, carry2):
                base0 = pl.multiple_of(t * 2 * _CHUNK, _CHUNK)
                base1 = base0 + _CHUNK
                dst0 = out_hbm.at[pl.ds(out_off + base0, _CHUNK)]
                dst1 = out_hbm.at[pl.ds(out_off + base1, _CHUNK)]

                @pl.when(t > 0)
                def _wait0():
                    pltpu.make_async_copy(out0_v, dst0, sem0).wait()

                gather_chunk(out0_v, base0)
                pltpu.async_copy(out0_v, dst0, sem0)

                @pl.when(t > 0)
                def _wait1():
                    pltpu.make_async_copy(out1_v, dst1, sem1).wait()

                gather_chunk(out1_v, base1)
                pltpu.async_copy(out1_v, dst1, sem1)
                return carry2

            lax.fori_loop(0, _NPAIR, pair, None)

            tail = pl.multiple_of(2 * _NPAIR * _CHUNK, _CHUNK)
            dst_t = out_hbm.at[pl.ds(out_off + tail, _CHUNK)]
            pltpu.make_async_copy(out0_v, dst_t, sem0).wait()
            pltpu.make_async_copy(out1_v, dst_t, sem1).wait()
            gather_chunk(out0_v, tail)
            pltpu.sync_copy(out0_v, dst_t)
            return carry

        lax.fori_loop(0, _BPW, do_batch, None)

    return gather_kernel


_gather = _make_kernel()


def kernel(data, x_indices, y_indices):
    B, H, W = data.shape
    x = x_indices.astype(jnp.int32)
    y = y_indices.astype(jnp.int32)
    out = _gather(x, y, data.reshape(B * H * W))
    return out.reshape(B, _N)


# speedup vs baseline: 3.2525x; 1.1699x over previous
import functools

import jax
import jax.numpy as jnp
from jax import lax
from jax.experimental import pallas as pl
from jax.experimental.pallas import tpu as pltpu
from jax.experimental.pallas import tpu_sc as plsc

_B, _H, _W = 256, 64, 1024
_N = 50000
_NW = 32
_BPW = _B // _NW
_CHUNK = 2000
_NCHUNK = _N // _CHUNK
_NPAIR = (_NCHUNK - 1) // 2
_VPC = _CHUNK // 16


def _make_kernel():
    mesh = plsc.VectorSubcoreMesh(core_axis_name="c", subcore_axis_name="s")

    @functools.partial(
        pl.kernel,
        mesh=mesh,
        out_type=jax.ShapeDtypeStruct((_B * _N,), jnp.float32),
        compiler_params=pltpu.CompilerParams(needs_layout_passes=False),
        scratch_types=[
            pltpu.VMEM((_N,), jnp.int32),
            pltpu.VMEM((_H, _W), jnp.float32),
            pltpu.VMEM((_CHUNK,), jnp.float32),
            pltpu.VMEM((_CHUNK,), jnp.float32),
            pltpu.VMEM((_CHUNK,), jnp.int32),
            pltpu.VMEM((_CHUNK,), jnp.int32),
            pltpu.SemaphoreType.DMA,
            pltpu.SemaphoreType.DMA,
        ],
    )
    def gather_kernel(x_hbm, y_hbm, data_hbm, out_hbm,
                      idx_v, row_v, out0_v, out1_v, xt_v, yt_v, sem0, sem1):
        wid = lax.axis_index("s") * 2 + lax.axis_index("c")

        def build_chunk(c, carry):
            base = pl.multiple_of(c * _CHUNK, _CHUNK)
            pltpu.sync_copy(x_hbm.at[pl.ds(base, _CHUNK)], xt_v)
            pltpu.sync_copy(y_hbm.at[pl.ds(base, _CHUNK)], yt_v)

            @plsc.parallel_loop(0, _VPC, unroll=8)
            def _build(j):
                xv = xt_v[pl.ds(j * 16, 16)]
                yv = yt_v[pl.ds(j * 16, 16)]
                idx_v[pl.ds(base + j * 16, 16)] = (xv << 10) | yv

            return carry

        lax.fori_loop(0, _NCHUNK, build_chunk, None)

        def gather_chunk(buf, cbase):
            @plsc.parallel_loop(0, _VPC, unroll=8)
            def _gather(j):
                pk = idx_v[pl.ds(cbase + j * 16, 16)]
                ix = pk >> 10
                iy = pk & 1023
                buf[pl.ds(j * 16, 16)] = plsc.load_gather(row_v, [ix, iy])

        def do_batch(i, carry):
            b = wid * _BPW + i
            out_off = pl.multiple_of(b * _N, 8)
            pltpu.sync_copy(data_hbm.at[b], row_v)

            def pair(t, carry2):
                base0 = pl.multiple_of(t * 2 * _CHUNK, _CHUNK)
                base1 = base0 + _CHUNK
                dst0 = out_hbm.at[pl.ds(out_off + base0, _CHUNK)]
                dst1 = out_hbm.at[pl.ds(out_off + base1, _CHUNK)]

                @pl.when(t > 0)
                def _wait0():
                    pltpu.make_async_copy(out0_v, dst0, sem0).wait()

                gather_chunk(out0_v, base0)
                pltpu.async_copy(out0_v, dst0, sem0)

                @pl.when(t > 0)
                def _wait1():
                    pltpu.make_async_copy(out1_v, dst1, sem1).wait()

                gather_chunk(out1_v, base1)
                pltpu.async_copy(out1_v, dst1, sem1)
                return carry2

            lax.fori_loop(0, _NPAIR, pair, None)

            tail = pl.multiple_of(2 * _NPAIR * _CHUNK, _CHUNK)
            dst_t = out_hbm.at[pl.ds(out_off + tail, _CHUNK)]
            pltpu.make_async_copy(out0_v, dst_t, sem0).wait()
            pltpu.make_async_copy(out1_v, dst_t, sem1).wait()
            gather_chunk(out0_v, tail)
            pltpu.sync_copy(out0_v, dst_t)
            return carry

        lax.fori_loop(0, _BPW, do_batch, None)

    return gather_kernel


_gather = _make_kernel()


def kernel(data, x_indices, y_indices):
    x = x_indices.astype(jnp.int32)
    y = y_indices.astype(jnp.int32)
    out = _gather(x, y, data)
    return out.reshape(_B, _N)


# speedup vs baseline: 3.5248x; 1.0837x over previous
import functools

import jax
import jax.numpy as jnp
from jax import lax
from jax.experimental import pallas as pl
from jax.experimental.pallas import tpu as pltpu
from jax.experimental.pallas import tpu_sc as plsc

_B, _H, _W = 256, 64, 1024
_N = 50000
_NW = 32
_BPW = _B // _NW
_CHUNK = 2000
_NCHUNK = _N // _CHUNK
_NPAIR = (_NCHUNK - 1) // 2
_VPC = _CHUNK // 16


def _make_kernel():
    mesh = plsc.VectorSubcoreMesh(core_axis_name="c", subcore_axis_name="s")

    @functools.partial(
        pl.kernel,
        mesh=mesh,
        out_type=jax.ShapeDtypeStruct((_B * _N,), jnp.float32),
        compiler_params=pltpu.CompilerParams(needs_layout_passes=False),
        scratch_types=[
            pltpu.VMEM((_N,), jnp.int32),
            pltpu.VMEM((_H, _W), jnp.float32),
            pltpu.VMEM((_CHUNK,), jnp.float32),
            pltpu.VMEM((_CHUNK,), jnp.float32),
            pltpu.VMEM((_CHUNK,), jnp.int32),
            pltpu.VMEM((_CHUNK,), jnp.int32),
            pltpu.VMEM((_CHUNK,), jnp.int32),
            pltpu.VMEM((_CHUNK,), jnp.int32),
            pltpu.SemaphoreType.DMA,
            pltpu.SemaphoreType.DMA,
            pltpu.SemaphoreType.DMA,
            pltpu.SemaphoreType.DMA,
            pltpu.SemaphoreType.DMA,
        ],
    )
    def gather_kernel(x_hbm, y_hbm, data_hbm, out_hbm,
                      idx_v, row_v, out0_v, out1_v,
                      xa_v, ya_v, xb_v, yb_v,
                      sem0, sem1, sema, semb, semr):
        wid = lax.axis_index("s") * 2 + lax.axis_index("c")
        b0 = wid * _BPW

        pltpu.async_copy(data_hbm.at[b0], row_v, semr)

        def start_load(c, xbuf, ybuf, sem):
            base = pl.multiple_of(c * _CHUNK, _CHUNK)
            pltpu.async_copy(x_hbm.at[pl.ds(base, _CHUNK)], xbuf, sem)
            pltpu.async_copy(y_hbm.at[pl.ds(base, _CHUNK)], ybuf, sem)

        def wait_load(c, xbuf, ybuf, sem):
            base = pl.multiple_of(c * _CHUNK, _CHUNK)
            pltpu.make_async_copy(x_hbm.at[pl.ds(base, _CHUNK)], xbuf, sem).wait()
            pltpu.make_async_copy(y_hbm.at[pl.ds(base, _CHUNK)], ybuf, sem).wait()

        def pack_chunk(c, xbuf, ybuf):
            base = pl.multiple_of(c * _CHUNK, _CHUNK)

            @plsc.parallel_loop(0, _VPC, unroll=8)
            def _build(j):
                xv = xbuf[pl.ds(j * 16, 16)]
                yv = ybuf[pl.ds(j * 16, 16)]
                idx_v[pl.ds(base + j * 16, 16)] = (xv << 10) | yv

        start_load(0, xa_v, ya_v, sema)

        def build_pair(t, carry):
            c0 = t * 2
            wait_load(c0, xa_v, ya_v, sema)
            start_load(c0 + 1, xb_v, yb_v, semb)
            pack_chunk(c0, xa_v, ya_v)
            wait_load(c0 + 1, xb_v, yb_v, semb)
            start_load(c0 + 2, xa_v, ya_v, sema)
            pack_chunk(c0 + 1, xb_v, yb_v)
            return carry

        lax.fori_loop(0, _NPAIR, build_pair, None)
        wait_load(_NCHUNK - 1, xa_v, ya_v, sema)
        pack_chunk(_NCHUNK - 1, xa_v, ya_v)

        def gather_chunk(buf, cbase):
            @plsc.parallel_loop(0, _VPC, unroll=8)
            def _gather(j):
                pk = idx_v[pl.ds(cbase + j * 16, 16)]
                ix = pk >> 10
                iy = pk & 1023
                buf[pl.ds(j * 16, 16)] = plsc.load_gather(row_v, [ix, iy])

        def do_batch(i, carry):
            b = b0 + i
            out_off = pl.multiple_of(b * _N, 8)
            pltpu.make_async_copy(data_hbm.at[b], row_v, semr).wait()

            def pair(t, carry2):
                base0 = pl.multiple_of(t * 2 * _CHUNK, _CHUNK)
                base1 = base0 + _CHUNK
                dst0 = out_hbm.at[pl.ds(out_off + base0, _CHUNK)]
                dst1 = out_hbm.at[pl.ds(out_off + base1, _CHUNK)]

                @pl.when(t > 0)
                def _wait0():
                    pltpu.make_async_copy(out0_v, dst0, sem0).wait()

                gather_chunk(out0_v, base0)
                pltpu.async_copy(out0_v, dst0, sem0)

                @pl.when(t > 0)
                def _wait1():
                    pltpu.make_async_copy(out1_v, dst1, sem1).wait()

                gather_chunk(out1_v, base1)
                pltpu.async_copy(out1_v, dst1, sem1)
                return carry2

            lax.fori_loop(0, _NPAIR, pair, None)

            tail = pl.multiple_of(2 * _NPAIR * _CHUNK, _CHUNK)
            dst_t = out_hbm.at[pl.ds(out_off + tail, _CHUNK)]
            pltpu.make_async_copy(out0_v, dst_t, sem0).wait()
            pltpu.make_async_copy(out1_v, dst_t, sem1).wait()
            gather_chunk(out0_v, tail)
            pltpu.async_copy(out0_v, dst_t, sem0)

            @pl.when(i + 1 < _BPW)
            def _prefetch():
                pltpu.async_copy(data_hbm.at[b + 1], row_v, semr)

            pltpu.make_async_copy(out0_v, dst_t, sem0).wait()
            return carry

        lax.fori_loop(0, _BPW, do_batch, None)

    return gather_kernel


_gather = _make_kernel()


def kernel(data, x_indices, y_indices):
    x = x_indices.astype(jnp.int32)
    y = y_indices.astype(jnp.int32)
    out = _gather(x, y, data)
    return out.reshape(_B, _N)
